# baseline (speedup 1.0000x reference)
import jax
import jax.numpy as jnp
from jax.experimental import pallas as pl
from jax.experimental.pallas import tpu as pltpu

_HIDDEN = 2880
_N_ORIG = 32
_N_COMP = 4
_N_TOTAL = 36
_TOP_K = 4
_INTER_C = 512

_TT = 256
_HCH = 720
_NH = _HIDDEN // _HCH


def _gu_body(x_ref, g_ref, u_ref, inter_ref, gacc, uacc):
    h = pl.program_id(2)

    @pl.when(h == 0)
    def _init():
        gacc[...] = jnp.zeros_like(gacc)
        uacc[...] = jnp.zeros_like(uacc)

    xt = x_ref[...]
    dn = (((0,), (0,)), ((), ()))
    gacc[...] += jax.lax.dot_general(
        xt, g_ref[0], dn, preferred_element_type=jnp.float32)
    uacc[...] += jax.lax.dot_general(
        xt, u_ref[0], dn, preferred_element_type=jnp.float32)

    @pl.when(h == _NH - 1)
    def _finish():
        g = gacc[...]
        inter_ref[0] = (g * jax.nn.sigmoid(g)) * uacc[...]


def _down_body(int_ref, w_ref, d_ref, o_ref):
    e = pl.program_id(2)

    @pl.when(e == 0)
    def _init():
        o_ref[...] = jnp.zeros_like(o_ref)

    wrow = w_ref[pl.ds(e % 8, 1), :]
    winter = wrow[0][:, None] * int_ref[0]
    o_ref[...] += jax.lax.dot_general(
        d_ref[0], winter, (((1,), (1,)), ((), ())),
        preferred_element_type=jnp.float32)


def _comp_body(x_ref, w_ref, w1_ref, b1_ref, w2_ref, b2_ref, o_ref):
    e = pl.program_id(1)

    @pl.when(e == 0)
    def _init():
        o_ref[...] = jnp.zeros_like(o_ref)

    x = x_ref[...]
    h = jnp.dot(x, w1_ref[0], preferred_element_type=jnp.float32)
    b1 = b1_ref[pl.ds(e, 1), :]
    h = jnp.maximum(h + b1, 0.0)
    eo = jnp.dot(h, w2_ref[0], preferred_element_type=jnp.float32)
    eo = eo + b2_ref[pl.ds(e, 1), :]
    wrow = w_ref[pl.ds(e, 1), :]
    o_ref[...] += wrow[0][:, None] * eo


@jax.jit
def _moe(flat, router_weight, router_bias, gate_up_proj, down_proj,
         comp_w1, comp_b1, comp_w2, comp_b2):
    t = flat.shape[0]
    logits = flat @ router_weight.T + router_bias
    topv, topi = jax.lax.top_k(logits, _TOP_K)
    wts = jax.nn.softmax(topv, axis=-1)
    w_full = jnp.zeros((t, _N_TOTAL), jnp.float32).at[
        jnp.arange(t)[:, None], topi].add(wts)
    w_sw = w_full[:, :_N_ORIG].T
    w_c = w_full[:, _N_ORIG:].T

    g_w = gate_up_proj[:, :, :_HIDDEN]
    u_w = gate_up_proj[:, :, _HIDDEN:]

    n_tt = t // _TT

    inter = pl.pallas_call(
        _gu_body,
        grid=(n_tt, _N_ORIG, _NH),
        in_specs=[
            pl.BlockSpec((_HCH, _TT), lambda tt, e, h: (h, tt)),
            pl.BlockSpec((1, _HCH, _HIDDEN), lambda tt, e, h: (e, h, 0)),
            pl.BlockSpec((1, _HCH, _HIDDEN), lambda tt, e, h: (e, h, 0)),
        ],
        out_specs=pl.BlockSpec((1, _TT, _HIDDEN), lambda tt, e, h: (e, tt, 0)),
        out_shape=jax.ShapeDtypeStruct((_N_ORIG, t, _HIDDEN), jnp.float32),
        scratch_shapes=[
            pltpu.VMEM((_TT, _HIDDEN), jnp.float32),
            pltpu.VMEM((_TT, _HIDDEN), jnp.float32),
        ],
    )(flat.T, g_w, u_w)

    out_sw_t = pl.pallas_call(
        _down_body,
        grid=(n_tt, _NH, _N_ORIG),
        in_specs=[
            pl.BlockSpec((1, _TT, _HIDDEN), lambda tt, dc, e: (e, tt, 0)),
            pl.BlockSpec((8, _TT), lambda tt, dc, e: (e // 8, tt)),
            pl.BlockSpec((1, _HCH, _HIDDEN), lambda tt, dc, e: (e, dc, 0)),
        ],
        out_specs=pl.BlockSpec((_HCH, _TT), lambda tt, dc, e: (dc, tt)),
        out_shape=jax.ShapeDtypeStruct((_HIDDEN, t), jnp.float32),
    )(inter, w_sw, down_proj)
    out_sw = out_sw_t.T

    out_c = pl.pallas_call(
        _comp_body,
        grid=(n_tt, _N_COMP),
        in_specs=[
            pl.BlockSpec((_TT, _HIDDEN), lambda tt, e: (tt, 0)),
            pl.BlockSpec((_N_COMP, _TT), lambda tt, e: (0, tt)),
            pl.BlockSpec((1, _HIDDEN, _INTER_C), lambda tt, e: (e, 0, 0)),
            pl.BlockSpec((_N_COMP, _INTER_C), lambda tt, e: (0, 0)),
            pl.BlockSpec((1, _INTER_C, _HIDDEN), lambda tt, e: (e, 0, 0)),
            pl.BlockSpec((_N_COMP, _HIDDEN), lambda tt, e: (0, 0)),
        ],
        out_specs=pl.BlockSpec((_TT, _HIDDEN), lambda tt, e: (tt, 0)),
        out_shape=jax.ShapeDtypeStruct((t, _HIDDEN), jnp.float32),
    )(flat, w_c, comp_w1, comp_b1, comp_w2, comp_b2)

    return out_sw + out_c


def kernel(hidden_states, router_weight, router_bias, gate_up_proj, down_proj,
           comp_w1, comp_b1, comp_w2, comp_b2):
    b, s, d = hidden_states.shape
    flat = hidden_states.reshape(-1, d)
    out = _moe(flat, router_weight, router_bias, gate_up_proj, down_proj,
               comp_w1, comp_b1, comp_w2, comp_b2)
    return out.reshape(b, s, d)
